# Initial kernel scaffold; baseline (speedup 1.0000x reference)
import jax
import jax.numpy as jnp
from jax.experimental import pallas as pl


def kernel(probs, labels):
    raise NotImplementedError("write your pallas kernel here")


# speedup vs baseline: 1.3021x; 1.3021x over previous
import functools

import jax
import jax.numpy as jnp
from jax import lax
from jax.experimental import pallas as pl
from jax.experimental.pallas import tpu as pltpu
from jax.experimental.pallas import tpu_sc as plsc

_NC = 2
_NS = 16
_NW = _NC * _NS
_L = 16

_N = 1_000_000
_C = 10
_NBINS = 15
_R = 2000
_NT = _N // _R
_PAD = 256
_ACC = _L * _PAD


def _sc_body(probs_hbm, labels_hbm, out_hbm, probs_v, labels_v, acc_v, red_v):
    cid = lax.axis_index("c")
    sid = lax.axis_index("s")
    wid = sid * _NC + cid

    lane = lax.iota(jnp.int32, _L)
    lane_pad = lane * _PAD
    lane_c = lane * _C
    zeros16 = jnp.zeros((_L,), jnp.float32)
    neg1 = jnp.full((_L,), -1.0, jnp.float32)

    clsok = []
    base_a = []
    for p in range(5):
        m = lane + 16 * p
        for kk in (80, 40, 20, 10):
            m = jnp.where(m >= kk, m - kk, m)
        clsok.append(m != 1)
        base_a.append(lane_pad + m * 16)

    def zero_body(k, _):
        acc_v[pl.ds(k * _L, _L)] = zeros16
        return 0

    lax.fori_loop(0, _ACC // _L, zero_body, 0)

    ntiles_w = (_NT - 1 - wid) // _NW + 1

    def tile_body(i, _):
        t = wid + i * _NW
        row0 = t * _R
        pltpu.sync_copy(probs_hbm.at[pl.ds(row0 * _C, _R * _C)], probs_v)
        pltpu.sync_copy(labels_hbm.at[pl.ds(row0, _R)], labels_v)

        def pass_a(g, _):
            base = g * 80
            for p in range(5):
                v = probs_v[pl.ds(base + p * _L, _L)]
                j = jnp.minimum((v * 15.0).astype(jnp.int32), _NBINS - 1)
                mask = (v > 0.0) & clsok[p]
                plsc.addupdate_scatter(acc_v, [base_a[p] + j], v, mask=mask)
            return 0

        lax.fori_loop(0, _R * _C // 80, pass_a, 0)

        def pass_b(m, _):
            lbl = labels_v[pl.ds(m * _L, _L)]
            vidx = m * (_L * _C) + lane_c + lbl
            v = plsc.load_gather(probs_v, [vidx])
            j = jnp.minimum((v * 15.0).astype(jnp.int32), _NBINS - 1)
            mask = (v > 0.0) & (lbl != 1)
            plsc.addupdate_scatter(acc_v, [lane_pad + lbl * 16 + j], neg1, mask=mask)
            return 0

        lax.fori_loop(0, _R // _L, pass_b, 0)
        return 0

    lax.fori_loop(0, ntiles_w, tile_body, 0)

    for k in range(_PAD // _L):
        s = acc_v[pl.ds(k * _L, _L)]
        for ln in range(1, _L):
            s = s + acc_v[pl.ds(ln * _PAD + k * _L, _L)]
        red_v[pl.ds(k * _L, _L)] = s
    pltpu.sync_copy(red_v, out_hbm.at[wid])


@functools.cache
def _get_sc_kernel():
    return pl.kernel(
        _sc_body,
        out_type=jax.ShapeDtypeStruct((_NW, _PAD), jnp.float32),
        mesh=plsc.VectorSubcoreMesh(
            core_axis_name="c", subcore_axis_name="s",
            num_cores=_NC, num_subcores=_NS,
        ),
        compiler_params=pltpu.CompilerParams(needs_layout_passes=False),
        scratch_types=[
            pltpu.VMEM((_R * _C,), jnp.float32),
            pltpu.VMEM((_R,), jnp.int32),
            pltpu.VMEM((_ACC,), jnp.float32),
            pltpu.VMEM((_PAD,), jnp.float32),
        ],
    )


def _combine_body(p_ref, o_ref):
    s = jnp.sum(p_ref[...], axis=0)
    o_ref[0, 0] = jnp.sum(jnp.abs(s)) * (1.0 / float(_C * _N))


_combine = pl.pallas_call(
    _combine_body,
    out_shape=jax.ShapeDtypeStruct((1, 1), jnp.float32),
    out_specs=pl.BlockSpec(memory_space=pltpu.SMEM),
)


@jax.jit
def kernel(probs, labels):
    partials = _get_sc_kernel()(probs.reshape(-1), labels)
    return _combine(partials)[0, 0]
